# baseline (speedup 1.0000x reference)
import functools

import jax
import jax.numpy as jnp
from jax import lax
from jax.experimental import pallas as pl
from jax.experimental.pallas import tpu as pltpu
from jax.experimental.pallas import tpu_sc as plsc

ROWS = 1024
COLS = 100000
K = 64

_info = plsc.get_sparse_core_info()
NC = _info.num_cores
NS = _info.num_subcores
NW = NC * NS
ROWS_PER_W = ROWS // NW
BR, BC = 16, 128

_mesh = plsc.VectorSubcoreMesh(core_axis_name="c", subcore_axis_name="s")


@functools.partial(
    pl.kernel,
    mesh=_mesh,
    out_type=jax.ShapeDtypeStruct((NW, BR, BC), jnp.float32),
    scratch_types=[
        pltpu.VMEM((BR, BC), jnp.int32),
        pltpu.VMEM((BR, BC), jnp.float32),
        pltpu.SemaphoreType.DMA,
    ],
)
def _gather_kernel(x_hbm, idx_hbm, out_hbm, idx_v, rows_v, sem):
    wid = lax.axis_index("s") * NC + lax.axis_index("c")
    pltpu.sync_copy(idx_hbm.at[wid], idx_v)
    wid_off = wid * (ROWS_PER_W * COLS)
    for r in range(BR):
        for c in range(BC // 16):
            off = wid_off + (2 * r + c // 4) * COLS
            sl = pl.ds(c * 16, 16)
            idx_v[r, sl] = idx_v[r, sl] + off
    cps = [
        pltpu.async_copy(x_hbm.at[idx_v.at[r]], rows_v.at[r], sem)
        for r in range(BR)
    ]
    for cp in cps:
        cp.wait()
    pltpu.sync_copy(rows_v, out_hbm.at[wid])


def kernel(x, index):
    x_flat = x.reshape(-1)
    idx_blocks = index.reshape(NW, BR, BC)
    out = _gather_kernel(x_flat, idx_blocks)
    return out.reshape(ROWS, K)


# speedup vs baseline: 1.8260x; 1.8260x over previous
import functools

import jax
import jax.numpy as jnp
from jax import lax
from jax.experimental import pallas as pl
from jax.experimental.pallas import tpu as pltpu
from jax.experimental.pallas import tpu_sc as plsc

ROWS = 1024
COLS = 100000
K = 64
N = ROWS * K

_info = plsc.get_sparse_core_info()
NC = _info.num_cores
NS = _info.num_subcores
NW = NC * NS
RPW = ROWS // NW
EPW = N // NW

NB = 32
NR = EPW // NB

_GDN = lax.GatherDimensionNumbers(
    offset_dims=(), collapsed_slice_dims=(0,), start_index_map=(0,)
)

_mesh = plsc.VectorSubcoreMesh(core_axis_name="c", subcore_axis_name="s")


@functools.partial(
    pl.kernel,
    mesh=_mesh,
    out_type=jax.ShapeDtypeStruct((N,), jnp.float32),
    scratch_types=[
        pltpu.VMEM((EPW,), jnp.int32),
        pltpu.VMEM((EPW,), jnp.float32),
        pltpu.VMEM((NB * 8, 128), jnp.float32),
        pltpu.VMEM((NB * 8, 128), jnp.float32),
        pltpu.SemaphoreType.DMA,
        pltpu.SemaphoreType.DMA,
    ],
)
def _gather_kernel(
    x_hbm, idx_hbm, out_hbm, idx_v, out_v, buf_a, buf_b, sem_a, sem_b
):
    wid = lax.axis_index("s") * NC + lax.axis_index("c")
    ebase = wid * EPW
    rowbase = wid * RPW
    pltpu.sync_copy(idx_hbm.at[pl.ds(ebase, EPW)], idx_v)

    lanes16 = lax.iota(jnp.int32, 16)

    def fire(n, buf, sem):
        row8 = pl.multiple_of((rowbase + (n >> 1)) & -8, 8)
        for h in range(2):
            jv = idx_v[pl.ds(n * NB + h * 16, 16)]
            cv = jv & -128
            for t in range(16):
                c128 = pl.multiple_of(cv[t], 128)
                pltpu.async_copy(
                    x_hbm.at[pl.ds(row8, 8), pl.ds(c128, 128)],
                    buf.at[pl.ds((h * 16 + t) * 8, 8)],
                    sem,
                )

    def drain(buf, sem):
        pltpu.make_async_copy(
            x_hbm.at[pl.ds(0, NB * 8), pl.ds(0, 128)], buf, sem
        ).wait()

    def extract(n, buf):
        subl = (n >> 1) & 7
        for h in range(2):
            jv = idx_v[pl.ds(n * NB + h * 16, 16)]
            acc = jnp.zeros((16,), jnp.float32)
            for t in range(16):
                s = jv[t]
                v2 = buf[(h * 16 + t) * 8 + subl, pl.ds(s & 112, 16)]
                lvec = jnp.full((16,), s & 15, jnp.int32)
                w = lax.gather(
                    v2,
                    lvec[:, None],
                    _GDN,
                    (1,),
                    mode=lax.GatherScatterMode.PROMISE_IN_BOUNDS,
                )
                acc = jnp.where(lanes16 == t, w, acc)
            out_v[pl.ds(n * NB + h * 16, 16)] = acc

    fire(0, buf_a, sem_a)
    fire(1, buf_b, sem_b)

    @pl.loop(0, NR // 2 - 1)
    def _pipeline(k):
        n0 = 2 * k
        drain(buf_a, sem_a)
        extract(n0, buf_a)
        fire(n0 + 2, buf_a, sem_a)
        drain(buf_b, sem_b)
        extract(n0 + 1, buf_b)
        fire(n0 + 3, buf_b, sem_b)

    drain(buf_a, sem_a)
    extract(NR - 2, buf_a)
    drain(buf_b, sem_b)
    extract(NR - 1, buf_b)

    pltpu.sync_copy(out_v, out_hbm.at[pl.ds(ebase, EPW)])


def kernel(x, index):
    out = _gather_kernel(x, index.reshape(N))
    return out.reshape(ROWS, K)
